# baseline (speedup 1.0000x reference)
import functools

import jax
import jax.numpy as jnp
from jax import lax
from jax.experimental import pallas as pl
from jax.experimental.pallas import tpu as pltpu
from jax.experimental.pallas import tpu_sc as plsc

N_NODES = 10000
N_EDGES = 160000
IN_DIM = 256
HIDDEN = 256
OUT_DIM = 128

NC = 2
NS = 16
NW = NC * NS

N_PAD = 10240
CHUNK = 128
NCHUNK_T = N_EDGES // CHUNK
NCHUNK = 40
NCHUNK_LAST = NCHUNK_T - 31 * NCHUNK
RPT = 624
RPT_LAST = N_NODES - 15 * RPT

_DENSE_BR = 1000


def _dense_body(f_ref, w0_ref, w1_ref, w2_ref, o_ref):
    h = jnp.dot(f_ref[...].astype(jnp.bfloat16),
                w0_ref[...].astype(jnp.bfloat16),
                preferred_element_type=jnp.float32)
    h = jnp.maximum(h, 0.0)
    h = jnp.dot(h.astype(jnp.bfloat16), w1_ref[...].astype(jnp.bfloat16),
                preferred_element_type=jnp.float32)
    h = jnp.maximum(h, 0.0)
    o_ref[...] = jnp.dot(h.astype(jnp.bfloat16),
                         w2_ref[...].astype(jnp.bfloat16),
                         preferred_element_type=jnp.float32)


def _dense(f, W0, W1, W2):
    grid = (N_NODES // _DENSE_BR,)
    return pl.pallas_call(
        _dense_body,
        grid=grid,
        in_specs=[
            pl.BlockSpec((_DENSE_BR, IN_DIM), lambda i: (i, 0)),
            pl.BlockSpec((IN_DIM, HIDDEN), lambda i: (0, 0)),
            pl.BlockSpec((HIDDEN, HIDDEN), lambda i: (0, 0)),
            pl.BlockSpec((HIDDEN, OUT_DIM), lambda i: (0, 0)),
        ],
        out_specs=pl.BlockSpec((_DENSE_BR, OUT_DIM), lambda i: (i, 0)),
        out_shape=jax.ShapeDtypeStruct((N_NODES, OUT_DIM), jnp.float32),
    )(f, W0, W1, W2)


def _combine_body(p_ref, o_ref):
    o_ref[...] = p_ref[0] + p_ref[1]


def _combine(p):
    grid = (N_NODES // _DENSE_BR,)
    return pl.pallas_call(
        _combine_body,
        grid=grid,
        in_specs=[pl.BlockSpec((NC, _DENSE_BR, OUT_DIM), lambda i: (0, i, 0))],
        out_specs=pl.BlockSpec((_DENSE_BR, OUT_DIM), lambda i: (i, 0)),
        out_shape=jax.ShapeDtypeStruct((N_NODES, OUT_DIM), jnp.float32),
    )(p)


_SC_MESH = plsc.VectorSubcoreMesh(core_axis_name="c", subcore_axis_name="s")


def _edge_pipeline(h_hbm, src_v, dst_v, rb0, rb1, acc, gs0, gs1, n):
    pltpu.async_copy(h_hbm.at[src_v.at[0]], rb0, gs0)
    pltpu.async_copy(h_hbm.at[src_v.at[1]], rb1, gs1)

    @pl.loop(0, n, step=2)
    def _(j):
        pltpu.make_async_copy(h_hbm.at[src_v.at[j]], rb0, gs0).wait()
        pltpu.sync_copy(rb0, acc.at[dst_v.at[j]], add=True)

        @pl.when(j + 2 < n)
        def _():
            pltpu.async_copy(h_hbm.at[src_v.at[j + 2]], rb0, gs0)

        pltpu.make_async_copy(h_hbm.at[src_v.at[j + 1]], rb1, gs1).wait()
        pltpu.sync_copy(rb1, acc.at[dst_v.at[j + 1]], add=True)

        @pl.when(j + 3 < n)
        def _():
            pltpu.async_copy(h_hbm.at[src_v.at[j + 3]], rb1, gs1)


@functools.partial(
    pl.kernel,
    out_type=jax.ShapeDtypeStruct((NC, N_NODES, OUT_DIM), jnp.float32),
    mesh=_SC_MESH,
    scratch_types=[
        pltpu.VMEM((NCHUNK, CHUNK), jnp.int32),
        pltpu.VMEM((NCHUNK, CHUNK), jnp.int32),
        pltpu.VMEM((CHUNK, OUT_DIM), jnp.float32),
        pltpu.VMEM((CHUNK, OUT_DIM), jnp.float32),
        pltpu.VMEM_SHARED((N_PAD, OUT_DIM), jnp.float32),
        pltpu.SemaphoreType.DMA,
        pltpu.SemaphoreType.DMA,
        pltpu.SemaphoreType.DMA,
        pltpu.SemaphoreType.DMA,
    ],
)
def _prop(h_hbm, src_hbm, dst_hbm, zeros_hbm, out_hbm,
          src_v, dst_v, rb0, rb1, acc, gs0, gs1, ss0, ss1):
    c = lax.axis_index("c")
    s = lax.axis_index("s")
    wid = c * NS + s

    @pl.when(s < 15)
    def _():
        pltpu.sync_copy(zeros_hbm.at[pl.ds(0, RPT)],
                        acc.at[pl.ds(s * RPT, RPT)])

    @pl.when(s == 15)
    def _():
        pltpu.sync_copy(zeros_hbm, acc.at[pl.ds(15 * RPT, RPT_LAST)])
    @pl.when(wid < NW - 1)
    def _():
        pltpu.sync_copy(src_hbm.at[pl.ds(wid * NCHUNK, NCHUNK)], src_v)
        pltpu.sync_copy(dst_hbm.at[pl.ds(wid * NCHUNK, NCHUNK)], dst_v)

    @pl.when(wid == NW - 1)
    def _():
        pltpu.sync_copy(src_hbm.at[pl.ds(31 * NCHUNK, NCHUNK_LAST)],
                        src_v.at[pl.ds(0, NCHUNK_LAST)])
        pltpu.sync_copy(dst_hbm.at[pl.ds(31 * NCHUNK, NCHUNK_LAST)],
                        dst_v.at[pl.ds(0, NCHUNK_LAST)])

    plsc.subcore_barrier()

    @pl.when(wid < NW - 1)
    def _():
        _edge_pipeline(h_hbm, src_v, dst_v, rb0, rb1, acc, gs0, gs1, NCHUNK)

    @pl.when(wid == NW - 1)
    def _():
        _edge_pipeline(h_hbm, src_v, dst_v, rb0, rb1, acc, gs0, gs1,
                       NCHUNK_LAST)

    plsc.subcore_barrier()

    @pl.when(s < 15)
    def _():
        pltpu.sync_copy(acc.at[pl.ds(s * RPT, RPT)],
                        out_hbm.at[c, pl.ds(s * RPT, RPT)])

    @pl.when(s == 15)
    def _():
        pltpu.sync_copy(acc.at[pl.ds(15 * RPT, RPT_LAST)],
                        out_hbm.at[c, pl.ds(15 * RPT, RPT_LAST)])


def kernel(features, edge_index, W0, W1, W2):
    src = edge_index[0].reshape(NCHUNK_T, CHUNK)
    dst = edge_index[1].reshape(NCHUNK_T, CHUNK)
    zeros = jnp.zeros((RPT_LAST, OUT_DIM), jnp.float32)

    h = _dense(features, W0, W1, W2)
    for _ in range(3):
        partials = _prop(h, src, dst, zeros)
        h = _combine(partials)
    return h


# speedup vs baseline: 1.0013x; 1.0013x over previous
import functools

import jax
import jax.numpy as jnp
from jax import lax
from jax.experimental import pallas as pl
from jax.experimental.pallas import tpu as pltpu
from jax.experimental.pallas import tpu_sc as plsc

N_NODES = 10000
N_EDGES = 160000
IN_DIM = 256
HIDDEN = 256
OUT_DIM = 128

NC = 2
NS = 16
NW = NC * NS

N_PAD = 10240
CHUNK = 128
NCHUNK_T = N_EDGES // CHUNK
NCHUNK = 40
NCHUNK_LAST = NCHUNK_T - 31 * NCHUNK
RPT = 624
RPT_LAST = N_NODES - 15 * RPT

_DENSE_BR = 1000


def _dense_body(f_ref, w0_ref, w1_ref, w2_ref, o_ref):
    h = jnp.dot(f_ref[...], w0_ref[...], preferred_element_type=jnp.float32)
    h = jnp.maximum(h, 0.0)
    h = jnp.dot(h, w1_ref[...], preferred_element_type=jnp.float32)
    h = jnp.maximum(h, 0.0)
    o_ref[...] = jnp.dot(h, w2_ref[...], preferred_element_type=jnp.float32)


def _dense(f, W0, W1, W2):
    grid = (N_NODES // _DENSE_BR,)
    return pl.pallas_call(
        _dense_body,
        grid=grid,
        in_specs=[
            pl.BlockSpec((_DENSE_BR, IN_DIM), lambda i: (i, 0)),
            pl.BlockSpec((IN_DIM, HIDDEN), lambda i: (0, 0)),
            pl.BlockSpec((HIDDEN, HIDDEN), lambda i: (0, 0)),
            pl.BlockSpec((HIDDEN, OUT_DIM), lambda i: (0, 0)),
        ],
        out_specs=pl.BlockSpec((_DENSE_BR, OUT_DIM), lambda i: (i, 0)),
        out_shape=jax.ShapeDtypeStruct((N_NODES, OUT_DIM), jnp.float32),
    )(f, W0, W1, W2)


def _combine_body(p_ref, o_ref):
    o_ref[...] = p_ref[0] + p_ref[1]


def _combine(p):
    grid = (N_NODES // _DENSE_BR,)
    return pl.pallas_call(
        _combine_body,
        grid=grid,
        in_specs=[pl.BlockSpec((NC, _DENSE_BR, OUT_DIM), lambda i: (0, i, 0))],
        out_specs=pl.BlockSpec((_DENSE_BR, OUT_DIM), lambda i: (i, 0)),
        out_shape=jax.ShapeDtypeStruct((N_NODES, OUT_DIM), jnp.float32),
    )(p)


_SC_MESH = plsc.VectorSubcoreMesh(core_axis_name="c", subcore_axis_name="s")


def _edge_pipeline(h_hbm, src_v, dst_v, bufs, sems, acc, n):
    nb = len(bufs)
    for b in range(nb):
        pltpu.async_copy(h_hbm.at[src_v.at[b]], bufs[b], sems[b])

    @pl.loop(0, n, step=nb)
    def _(j):
        for b in range(nb):
            pltpu.make_async_copy(h_hbm.at[src_v.at[j + b]], bufs[b],
                                  sems[b]).wait()
            pltpu.sync_copy(bufs[b], acc.at[dst_v.at[j + b]], add=True)

            @pl.when(j + nb + b < n)
            def _(b=b):
                pltpu.async_copy(h_hbm.at[src_v.at[j + nb + b]], bufs[b],
                                 sems[b])


@functools.partial(
    pl.kernel,
    out_type=jax.ShapeDtypeStruct((NC, N_NODES, OUT_DIM), jnp.float32),
    mesh=_SC_MESH,
    scratch_types=[
        pltpu.VMEM((NCHUNK, CHUNK), jnp.int32),
        pltpu.VMEM((NCHUNK, CHUNK), jnp.int32),
        pltpu.VMEM((CHUNK, OUT_DIM), jnp.float32),
        pltpu.VMEM((CHUNK, OUT_DIM), jnp.float32),
        pltpu.VMEM_SHARED((N_NODES, OUT_DIM), jnp.float32),
        pltpu.SemaphoreType.DMA,
        pltpu.SemaphoreType.DMA,
    ],
)
def _prop(h_hbm, src_hbm, dst_hbm, zeros_hbm, out_hbm,
          src_v, dst_v, rb0, rb1, acc, gs0, gs1):
    c = lax.axis_index("c")
    s = lax.axis_index("s")
    wid = c * NS + s

    @pl.when(s < 15)
    def _():
        pltpu.sync_copy(zeros_hbm.at[pl.ds(0, RPT)],
                        acc.at[pl.ds(s * RPT, RPT)])

    @pl.when(s == 15)
    def _():
        pltpu.sync_copy(zeros_hbm, acc.at[pl.ds(15 * RPT, RPT_LAST)])
    @pl.when(wid < NW - 1)
    def _():
        pltpu.sync_copy(src_hbm.at[pl.ds(wid * NCHUNK, NCHUNK)], src_v)
        pltpu.sync_copy(dst_hbm.at[pl.ds(wid * NCHUNK, NCHUNK)], dst_v)

    @pl.when(wid == NW - 1)
    def _():
        pltpu.sync_copy(src_hbm.at[pl.ds(31 * NCHUNK, NCHUNK_LAST)],
                        src_v.at[pl.ds(0, NCHUNK_LAST)])
        pltpu.sync_copy(dst_hbm.at[pl.ds(31 * NCHUNK, NCHUNK_LAST)],
                        dst_v.at[pl.ds(0, NCHUNK_LAST)])

    plsc.subcore_barrier()

    bufs = (rb0, rb1)
    sems = (gs0, gs1)

    @pl.when(wid < NW - 1)
    def _():
        _edge_pipeline(h_hbm, src_v, dst_v, bufs, sems, acc, NCHUNK)

    @pl.when(wid == NW - 1)
    def _():
        _edge_pipeline(h_hbm, src_v, dst_v, bufs, sems, acc, NCHUNK_LAST)

    plsc.subcore_barrier()

    @pl.when(s < 15)
    def _():
        pltpu.sync_copy(acc.at[pl.ds(s * RPT, RPT)],
                        out_hbm.at[c, pl.ds(s * RPT, RPT)])

    @pl.when(s == 15)
    def _():
        pltpu.sync_copy(acc.at[pl.ds(15 * RPT, RPT_LAST)],
                        out_hbm.at[c, pl.ds(15 * RPT, RPT_LAST)])


def kernel(features, edge_index, W0, W1, W2):
    src = edge_index[0].reshape(NCHUNK_T, CHUNK)
    dst = edge_index[1].reshape(NCHUNK_T, CHUNK)
    zeros = jnp.zeros((RPT_LAST, OUT_DIM), jnp.float32)

    h = _dense(features, W0, W1, W2)
    for _ in range(3):
        partials = _prop(h, src, dst, zeros)
        h = _combine(partials)
    return h


# speedup vs baseline: 1.0241x; 1.0228x over previous
import functools

import jax
import jax.numpy as jnp
from jax import lax
from jax.experimental import pallas as pl
from jax.experimental.pallas import tpu as pltpu
from jax.experimental.pallas import tpu_sc as plsc

N_NODES = 10000
N_EDGES = 160000
IN_DIM = 256
HIDDEN = 256
OUT_DIM = 128

NC = 2
NS = 16
NW = NC * NS

N_PAD = 10240
CHUNK = 64
NCHUNK_T = N_EDGES // CHUNK
NCHUNK = 80
NCHUNK_LAST = NCHUNK_T - 31 * NCHUNK
PHASE = 40
RPT = 624
RPT_LAST = N_NODES - 15 * RPT

_DENSE_BR = 1000


def _dense_body(f_ref, w0_ref, w1_ref, w2_ref, o_ref):
    h = jnp.dot(f_ref[...], w0_ref[...], preferred_element_type=jnp.float32)
    h = jnp.maximum(h, 0.0)
    h = jnp.dot(h, w1_ref[...], preferred_element_type=jnp.float32)
    h = jnp.maximum(h, 0.0)
    o_ref[...] = jnp.dot(h, w2_ref[...], preferred_element_type=jnp.float32)


def _dense(f, W0, W1, W2):
    grid = (N_NODES // _DENSE_BR,)
    return pl.pallas_call(
        _dense_body,
        grid=grid,
        in_specs=[
            pl.BlockSpec((_DENSE_BR, IN_DIM), lambda i: (i, 0)),
            pl.BlockSpec((IN_DIM, HIDDEN), lambda i: (0, 0)),
            pl.BlockSpec((HIDDEN, HIDDEN), lambda i: (0, 0)),
            pl.BlockSpec((HIDDEN, OUT_DIM), lambda i: (0, 0)),
        ],
        out_specs=pl.BlockSpec((_DENSE_BR, OUT_DIM), lambda i: (i, 0)),
        out_shape=jax.ShapeDtypeStruct((N_NODES, OUT_DIM), jnp.float32),
    )(f, W0, W1, W2)


def _combine_body(p_ref, o_ref):
    o_ref[...] = p_ref[0] + p_ref[1]


def _combine(p):
    grid = (N_NODES // _DENSE_BR,)
    return pl.pallas_call(
        _combine_body,
        grid=grid,
        in_specs=[pl.BlockSpec((NC, _DENSE_BR, OUT_DIM), lambda i: (0, i, 0))],
        out_specs=pl.BlockSpec((_DENSE_BR, OUT_DIM), lambda i: (i, 0)),
        out_shape=jax.ShapeDtypeStruct((N_NODES, OUT_DIM), jnp.float32),
    )(p)


_SC_MESH = plsc.VectorSubcoreMesh(core_axis_name="c", subcore_axis_name="s")


def _edge_pipeline(h_hbm, src_v, dst_v, bufs, sems, acc, n):
    nb = len(bufs)
    for b in range(nb):
        pltpu.async_copy(h_hbm.at[src_v.at[b]], bufs[b], sems[b])

    @pl.loop(0, n, step=nb)
    def _(j):
        for b in range(nb):
            pltpu.make_async_copy(h_hbm.at[src_v.at[j + b]], bufs[b],
                                  sems[b]).wait()
            pltpu.sync_copy(bufs[b], acc.at[dst_v.at[j + b]], add=True)

            @pl.when(j + nb + b < n)
            def _(b=b):
                pltpu.async_copy(h_hbm.at[src_v.at[j + nb + b]], bufs[b],
                                 sems[b])


@functools.partial(
    pl.kernel,
    out_type=jax.ShapeDtypeStruct((NC, N_NODES, OUT_DIM), jnp.float32),
    mesh=_SC_MESH,
    scratch_types=[
        pltpu.VMEM((PHASE, CHUNK), jnp.int32),
        pltpu.VMEM((PHASE, CHUNK), jnp.int32),
        pltpu.VMEM((CHUNK, OUT_DIM), jnp.float32),
        pltpu.VMEM((CHUNK, OUT_DIM), jnp.float32),
        pltpu.VMEM((CHUNK, OUT_DIM), jnp.float32),
        pltpu.VMEM((CHUNK, OUT_DIM), jnp.float32),
        pltpu.VMEM_SHARED((N_NODES, OUT_DIM), jnp.float32),
        pltpu.SemaphoreType.DMA,
        pltpu.SemaphoreType.DMA,
        pltpu.SemaphoreType.DMA,
        pltpu.SemaphoreType.DMA,
    ],
)
def _prop(h_hbm, src_hbm, dst_hbm, zeros_hbm, out_hbm,
          src_v, dst_v, rb0, rb1, rb2, rb3, acc, gs0, gs1, gs2, gs3):
    c = lax.axis_index("c")
    s = lax.axis_index("s")
    wid = c * NS + s

    @pl.when(s < 15)
    def _():
        pltpu.sync_copy(zeros_hbm.at[pl.ds(0, RPT)],
                        acc.at[pl.ds(s * RPT, RPT)])

    @pl.when(s == 15)
    def _():
        pltpu.sync_copy(zeros_hbm, acc.at[pl.ds(15 * RPT, RPT_LAST)])
    @pl.when(wid < NW - 1)
    def _():
        pltpu.sync_copy(src_hbm.at[pl.ds(wid * NCHUNK, PHASE)], src_v)
        pltpu.sync_copy(dst_hbm.at[pl.ds(wid * NCHUNK, PHASE)], dst_v)

    @pl.when(wid == NW - 1)
    def _():
        pltpu.sync_copy(src_hbm.at[pl.ds(31 * NCHUNK, NCHUNK_LAST)],
                        src_v.at[pl.ds(0, NCHUNK_LAST)])
        pltpu.sync_copy(dst_hbm.at[pl.ds(31 * NCHUNK, NCHUNK_LAST)],
                        dst_v.at[pl.ds(0, NCHUNK_LAST)])

    plsc.subcore_barrier()

    bufs = (rb0, rb1, rb2, rb3)
    sems = (gs0, gs1, gs2, gs3)

    @pl.when(wid < NW - 1)
    def _():
        _edge_pipeline(h_hbm, src_v, dst_v, bufs, sems, acc, PHASE)
        pltpu.sync_copy(src_hbm.at[pl.ds(wid * NCHUNK + PHASE, PHASE)],
                        src_v)
        pltpu.sync_copy(dst_hbm.at[pl.ds(wid * NCHUNK + PHASE, PHASE)],
                        dst_v)
        _edge_pipeline(h_hbm, src_v, dst_v, bufs, sems, acc, PHASE)

    @pl.when(wid == NW - 1)
    def _():
        _edge_pipeline(h_hbm, src_v, dst_v, bufs, sems, acc, NCHUNK_LAST)

    plsc.subcore_barrier()

    @pl.when(s < 15)
    def _():
        pltpu.sync_copy(acc.at[pl.ds(s * RPT, RPT)],
                        out_hbm.at[c, pl.ds(s * RPT, RPT)])

    @pl.when(s == 15)
    def _():
        pltpu.sync_copy(acc.at[pl.ds(15 * RPT, RPT_LAST)],
                        out_hbm.at[c, pl.ds(15 * RPT, RPT_LAST)])


def kernel(features, edge_index, W0, W1, W2):
    src = edge_index[0].reshape(NCHUNK_T, CHUNK)
    dst = edge_index[1].reshape(NCHUNK_T, CHUNK)
    zeros = jnp.zeros((RPT_LAST, OUT_DIM), jnp.float32)

    h = _dense(features, W0, W1, W2)
    for _ in range(3):
        partials = _prop(h, src, dst, zeros)
        h = _combine(partials)
    return h
